# Initial kernel scaffold; baseline (speedup 1.0000x reference)
import functools

import numpy as np
import jax
import jax.numpy as jnp
from jax import lax
from jax.experimental import pallas as pl
from jax.experimental.pallas import tpu as pltpu
from jax.experimental.pallas import tpu_sc as plsc

LVLS = 16
TSIZE = 2 ** 19
HMASK = TSIZE - 1
P1, P2, P3 = 2654435761, 805459861, 3674653429
RES_LIST = [int(np.floor(16.0 * (1.5 ** l))) for l in range(LVLS)]

N = 524288
NC, NS, LANES = 2, 16, 16
NW = NC * NS
PER_TILE = N // NW
C = 1024
NCHUNK = PER_TILE // C
NG = C // LANES
K = 16 * C


def _enc_body(xyzt_hbm, tabs_hbm, res_hbm, enc_hbm,
              xbuf, resv, idx_buf, w_buf, gbuf, acc_buf, sem):
    wid = lax.axis_index("s") * NC + lax.axis_index("c")
    tile_base = wid * PER_TILE
    lanes = lax.iota(jnp.int32, LANES)
    zeros16 = jnp.zeros((LANES,), jnp.int32)
    ones16 = jnp.ones((LANES,), jnp.int32)

    pltpu.sync_copy(res_hbm, resv)

    def chunk_body(ci, _):
        base = tile_base + ci * C
        pltpu.sync_copy(xyzt_hbm.at[:, pl.ds(base, C)], xbuf)

        def level_body(l, _):
            res = plsc.load_gather(resv, [jnp.full((LANES,), l, jnp.int32)])
            ltT = l * TSIZE

            def hash_group(g, _):
                gb = g * LANES
                x = xbuf[0, pl.ds(gb, LANES)]
                y = xbuf[1, pl.ds(gb, LANES)]
                z = xbuf[2, pl.ds(gb, LANES)]
                t = xbuf[3, pl.ds(gb, LANES)]
                hs = []
                ws = []
                for coord, prime in ((x, 1), (y, P1), (z, P2), (t, P3)):
                    pos = coord * res
                    p0 = pos.astype(jnp.int32)
                    frac = pos - p0.astype(jnp.float32)
                    u0 = p0.astype(jnp.uint32)
                    pj = jnp.uint32(prime)
                    h0 = u0 * pj
                    hs.append((h0, h0 + pj))
                    ws.append((1.0 - frac, frac))
                hxy = [[hs[0][a] ^ hs[1][b] for b in (0, 1)] for a in (0, 1)]
                hzt = [[hs[2][a] ^ hs[3][b] for b in (0, 1)] for a in (0, 1)]
                wxy = [[ws[0][a] * ws[1][b] for b in (0, 1)] for a in (0, 1)]
                wzt = [[ws[2][a] * ws[3][b] for b in (0, 1)] for a in (0, 1)]
                for c in range(16):
                    bx, by, bz, bt = c & 1, (c >> 1) & 1, (c >> 2) & 1, (c >> 3) & 1
                    hh = (hxy[bx][by] ^ hzt[bz][bt]) & jnp.uint32(HMASK)
                    idx_buf[pl.ds(c * C + gb, LANES)] = hh.astype(jnp.int32) + ltT
                    w_buf[pl.ds(c * C + gb, LANES)] = wxy[bx][by] * wzt[bz][bt]
                return 0

            lax.fori_loop(0, NG, hash_group, 0)

            pltpu.async_copy(tabs_hbm.at[idx_buf], gbuf, sem).wait()

            def acc_group(g, _):
                gb = g * LANES
                acc0 = jnp.zeros((LANES,), jnp.float32)
                acc1 = jnp.zeros((LANES,), jnp.float32)
                for c in range(16):
                    rows = c * C + gb + lanes
                    g0 = plsc.load_gather(gbuf, [rows, zeros16])
                    g1 = plsc.load_gather(gbuf, [rows, ones16])
                    w = w_buf[pl.ds(c * C + gb, LANES)]
                    acc0 = acc0 + g0 * w
                    acc1 = acc1 + g1 * w
                acc_buf[2 * l, pl.ds(gb, LANES)] = acc0
                acc_buf[2 * l + 1, pl.ds(gb, LANES)] = acc1
                return 0

            lax.fori_loop(0, NG, acc_group, 0)
            return 0

        lax.fori_loop(0, LVLS, level_body, 0)
        pltpu.sync_copy(acc_buf, enc_hbm.at[:, pl.ds(base, C)])
        return 0

    lax.fori_loop(0, NCHUNK, chunk_body, 0)


@jax.jit
def _encode(xyzt_t, tabs, resarr):
    mesh = plsc.VectorSubcoreMesh(core_axis_name="c", subcore_axis_name="s")
    f = pl.kernel(
        _enc_body,
        out_type=jax.ShapeDtypeStruct((2 * LVLS, N), jnp.float32),
        mesh=mesh,
        scratch_types=[
            pltpu.VMEM((4, C), jnp.float32),
            pltpu.VMEM((LANES,), jnp.float32),
            pltpu.VMEM((K,), jnp.int32),
            pltpu.VMEM((K,), jnp.float32),
            pltpu.VMEM((K, 2), jnp.float32),
            pltpu.VMEM((2 * LVLS, C), jnp.float32),
            pltpu.SemaphoreType.DMA,
        ],
    )
    return f(xyzt_t, tabs, resarr)


def _mlp_body(enc_ref, w1_ref, w2_ref, w3_ref, out_ref):
    e = enc_ref[...]
    h = jnp.maximum(lax.dot_general(e, w1_ref[...],
                                    (((0,), (0,)), ((), ()))), 0.0)
    h = jnp.maximum(jnp.dot(h, w2_ref[...]), 0.0)
    out_ref[...] = jnp.dot(h, w3_ref[...])


@jax.jit
def _mlp(enc, W1, W2, W3):
    BN = 4096
    return pl.pallas_call(
        _mlp_body,
        grid=(N // BN,),
        in_specs=[
            pl.BlockSpec((2 * LVLS, BN), lambda i: (0, i)),
            pl.BlockSpec((2 * LVLS, 64), lambda i: (0, 0)),
            pl.BlockSpec((64, 64), lambda i: (0, 0)),
            pl.BlockSpec((64, 1), lambda i: (0, 0)),
        ],
        out_specs=pl.BlockSpec((BN, 1), lambda i: (i, 0)),
        out_shape=jax.ShapeDtypeStruct((N, 1), jnp.float32),
    )(enc, W1, W2, W3)


def kernel(xyz, t, tables, W1, W2, W3):
    xyzt_t = jnp.concatenate([xyz, t], axis=1).T
    tabs = tables.reshape(LVLS * TSIZE, 2)
    resarr = jnp.array(RES_LIST, dtype=jnp.float32)
    enc = _encode(xyzt_t, tabs, resarr)
    return _mlp(enc, W1, W2, W3)


# speedup vs baseline: 45.6550x; 45.6550x over previous
import numpy as np
import jax
import jax.numpy as jnp
from jax import lax
from jax.experimental import pallas as pl
from jax.experimental.pallas import tpu as pltpu
from jax.experimental.pallas import tpu_sc as plsc

LVLS = 16
TSIZE = 2 ** 19
HMASK = TSIZE - 1
P1, P2, P3 = 2654435761, 805459861, 3674653429

N = 524288
NC, NS, LANES = 2, 16, 16
NW = NC * NS
PER_TILE = N // NW
C = 512
NCHUNK = PER_TILE // C
NG = (2 * C) // LANES
K = 32 * C


def _enc_body(xyzt_hbm, tabs_hbm, enc_hbm, xbuf, idx_buf, w_buf, gbuf,
              acc_buf, sem):
    wid = lax.axis_index("s") * NC + lax.axis_index("c")
    tile_base = wid * PER_TILE
    alt01 = lax.iota(jnp.int32, LANES) & 1

    def chunk_body(ci, _):
        base = 2 * (tile_base + ci * C)
        pltpu.sync_copy(xyzt_hbm.at[:, pl.ds(base, 2 * C)], xbuf)

        def level_body(l, carry):
            rf, altlt = carry
            res = rf.astype(jnp.int32).astype(jnp.float32)

            def hash_group(g, _):
                gb = g * LANES
                x = xbuf[0, pl.ds(gb, LANES)]
                y = xbuf[1, pl.ds(gb, LANES)]
                z = xbuf[2, pl.ds(gb, LANES)]
                t = xbuf[3, pl.ds(gb, LANES)]
                hs = []
                ws = []
                for coord, prime in ((x, 1), (y, P1), (z, P2), (t, P3)):
                    pos = coord * res
                    p0 = pos.astype(jnp.int32)
                    frac = pos - p0.astype(jnp.float32)
                    u0 = p0.astype(jnp.uint32)
                    pj = jnp.uint32(prime)
                    h0 = u0 * pj
                    hs.append((h0, h0 + pj))
                    ws.append((1.0 - frac, frac))
                hxy = [[hs[0][a] ^ hs[1][b] for b in (0, 1)] for a in (0, 1)]
                hzt = [[hs[2][a] ^ hs[3][b] for b in (0, 1)] for a in (0, 1)]
                wxy = [[ws[0][a] * ws[1][b] for b in (0, 1)] for a in (0, 1)]
                wzt = [[ws[2][a] * ws[3][b] for b in (0, 1)] for a in (0, 1)]
                for c in range(16):
                    bx, by, bz, bt = c & 1, (c >> 1) & 1, (c >> 2) & 1, (c >> 3) & 1
                    hh = (hxy[bx][by] ^ hzt[bz][bt]) & jnp.uint32(HMASK)
                    eidx = (hh.astype(jnp.int32) << 1) + altlt
                    idx_buf[pl.ds(c * 2 * C + gb, LANES)] = eidx
                    w_buf[pl.ds(c * 2 * C + gb, LANES)] = wxy[bx][by] * wzt[bz][bt]
                return 0

            lax.fori_loop(0, NG, hash_group, 0)

            pltpu.async_copy(tabs_hbm.at[idx_buf], gbuf, sem).wait()

            def acc_group(g, _):
                gb = g * LANES
                acc = jnp.zeros((LANES,), jnp.float32)
                for c in range(16):
                    gv = gbuf[pl.ds(c * 2 * C + gb, LANES)]
                    wv = w_buf[pl.ds(c * 2 * C + gb, LANES)]
                    acc = acc + gv * wv
                acc_buf[l, pl.ds(gb, LANES)] = acc
                return 0

            lax.fori_loop(0, NG, acc_group, 0)
            return rf * 1.5, altlt + 2 * TSIZE

        rf0 = jnp.full((LANES,), 16.0, jnp.float32)
        lax.fori_loop(0, LVLS, level_body, (rf0, alt01))
        pltpu.sync_copy(acc_buf, enc_hbm.at[:, pl.ds(base, 2 * C)])
        return 0

    lax.fori_loop(0, NCHUNK, chunk_body, 0)


@jax.jit
def _encode(xyzt_dup, tabs_flat):
    mesh = plsc.VectorSubcoreMesh(core_axis_name="c", subcore_axis_name="s")
    f = pl.kernel(
        _enc_body,
        out_type=jax.ShapeDtypeStruct((LVLS, 2 * N), jnp.float32),
        mesh=mesh,
        scratch_types=[
            pltpu.VMEM((4, 2 * C), jnp.float32),
            pltpu.VMEM((K,), jnp.int32),
            pltpu.VMEM((K,), jnp.float32),
            pltpu.VMEM((K,), jnp.float32),
            pltpu.VMEM((LVLS, 2 * C), jnp.float32),
            pltpu.SemaphoreType.DMA,
        ],
    )
    return f(xyzt_dup, tabs_flat)


def _mlp_body(enc_ref, w1_ref, w2_ref, w3_ref, out_ref):
    h = jnp.maximum(jnp.dot(enc_ref[...], w1_ref[...]), 0.0)
    h = jnp.maximum(jnp.dot(h, w2_ref[...]), 0.0)
    out_ref[...] = jnp.dot(h, w3_ref[...])


@jax.jit
def _mlp(enc, W1, W2, W3):
    BN = 4096
    return pl.pallas_call(
        _mlp_body,
        grid=(N // BN,),
        in_specs=[
            pl.BlockSpec((BN, 2 * LVLS), lambda i: (i, 0)),
            pl.BlockSpec((2 * LVLS, 64), lambda i: (0, 0)),
            pl.BlockSpec((64, 64), lambda i: (0, 0)),
            pl.BlockSpec((64, 1), lambda i: (0, 0)),
        ],
        out_specs=pl.BlockSpec((BN, 1), lambda i: (i, 0)),
        out_shape=jax.ShapeDtypeStruct((N, 1), jnp.float32),
    )(enc, W1, W2, W3)


def kernel(xyz, t, tables, W1, W2, W3):
    xyzt_t = jnp.concatenate([xyz, t], axis=1).T
    xyzt_dup = jnp.repeat(xyzt_t, 2, axis=1)
    tabs_flat = tables.reshape(LVLS * TSIZE * 2)
    enc = _encode(xyzt_dup, tabs_flat)
    encN = enc.reshape(LVLS, N, 2).transpose(1, 0, 2).reshape(N, 2 * LVLS)
    return _mlp(encN, W1, W2, W3)


# speedup vs baseline: 98.9026x; 2.1663x over previous
import numpy as np
import jax
import jax.numpy as jnp
from jax import lax
from jax.experimental import pallas as pl
from jax.experimental.pallas import tpu as pltpu
from jax.experimental.pallas import tpu_sc as plsc

LVLS = 16
TSIZE = 2 ** 19
HMASK = TSIZE - 1
P1, P2, P3 = 2654435761, 805459861, 3674653429

N = 524288
NC, NS, LANES = 2, 16, 16
NW = NC * NS
PER_TILE = N // NW
C = 1024
NCHUNK = PER_TILE // C
NG = C // LANES
K = 16 * C


def _enc_body(xyzt_hbm, tab0_hbm, tab1_hbm, enc_hbm,
              xbuf, idx_buf, w_buf, g0buf, g1buf, acc_buf, sem):
    wid = lax.axis_index("s") * NC + lax.axis_index("c")
    tile_base = wid * PER_TILE

    def chunk_body(ci, _):
        base = tile_base + ci * C
        pltpu.sync_copy(xyzt_hbm.at[:, pl.ds(base, C)], xbuf)

        def level_body(l, carry):
            rf, ltv = carry
            res = rf.astype(jnp.int32).astype(jnp.float32)

            def hash_group(g, _):
                gb = g * LANES
                x = xbuf[0, pl.ds(gb, LANES)]
                y = xbuf[1, pl.ds(gb, LANES)]
                z = xbuf[2, pl.ds(gb, LANES)]
                t = xbuf[3, pl.ds(gb, LANES)]
                hs = []
                ws = []
                for coord, prime in ((x, 1), (y, P1), (z, P2), (t, P3)):
                    pos = coord * res
                    p0 = pos.astype(jnp.int32)
                    frac = pos - p0.astype(jnp.float32)
                    u0 = p0.astype(jnp.uint32)
                    pj = jnp.uint32(prime)
                    h0 = u0 * pj
                    hs.append((h0, h0 + pj))
                    ws.append((1.0 - frac, frac))
                hxy = [[hs[0][a] ^ hs[1][b] for b in (0, 1)] for a in (0, 1)]
                hzt = [[hs[2][a] ^ hs[3][b] for b in (0, 1)] for a in (0, 1)]
                wxy = [[ws[0][a] * ws[1][b] for b in (0, 1)] for a in (0, 1)]
                wzt = [[ws[2][a] * ws[3][b] for b in (0, 1)] for a in (0, 1)]
                for c in range(16):
                    bx, by, bz, bt = c & 1, (c >> 1) & 1, (c >> 2) & 1, (c >> 3) & 1
                    hh = (hxy[bx][by] ^ hzt[bz][bt]) & jnp.uint32(HMASK)
                    idx_buf[pl.ds(c * C + gb, LANES)] = hh.astype(jnp.int32) + ltv
                    w_buf[pl.ds(c * C + gb, LANES)] = wxy[bx][by] * wzt[bz][bt]
                return 0

            lax.fori_loop(0, NG, hash_group, 0)

            cp0 = pltpu.async_copy(tab0_hbm.at[idx_buf], g0buf, sem)
            cp1 = pltpu.async_copy(tab1_hbm.at[idx_buf], g1buf, sem)
            cp0.wait()
            cp1.wait()

            def acc_group(g, _):
                gb = g * LANES
                acc0 = jnp.zeros((LANES,), jnp.float32)
                acc1 = jnp.zeros((LANES,), jnp.float32)
                for c in range(16):
                    wv = w_buf[pl.ds(c * C + gb, LANES)]
                    acc0 = acc0 + g0buf[pl.ds(c * C + gb, LANES)] * wv
                    acc1 = acc1 + g1buf[pl.ds(c * C + gb, LANES)] * wv
                acc_buf[2 * l, pl.ds(gb, LANES)] = acc0
                acc_buf[2 * l + 1, pl.ds(gb, LANES)] = acc1
                return 0

            lax.fori_loop(0, NG, acc_group, 0)
            return rf * 1.5, ltv + TSIZE

        rf0 = jnp.full((LANES,), 16.0, jnp.float32)
        ltv0 = jnp.zeros((LANES,), jnp.int32)
        lax.fori_loop(0, LVLS, level_body, (rf0, ltv0))
        pltpu.sync_copy(acc_buf, enc_hbm.at[:, pl.ds(base, C)])
        return 0

    lax.fori_loop(0, NCHUNK, chunk_body, 0)


@jax.jit
def _encode(xyzt_t, tab0, tab1):
    mesh = plsc.VectorSubcoreMesh(core_axis_name="c", subcore_axis_name="s")
    f = pl.kernel(
        _enc_body,
        out_type=jax.ShapeDtypeStruct((2 * LVLS, N), jnp.float32),
        mesh=mesh,
        scratch_types=[
            pltpu.VMEM((4, C), jnp.float32),
            pltpu.VMEM((K,), jnp.int32),
            pltpu.VMEM((K,), jnp.float32),
            pltpu.VMEM((K,), jnp.float32),
            pltpu.VMEM((K,), jnp.float32),
            pltpu.VMEM((2 * LVLS, C), jnp.float32),
            pltpu.SemaphoreType.DMA,
        ],
    )
    return f(xyzt_t, tab0, tab1)


def _mlp_body(enc_ref, w1_ref, w2_ref, w3_ref, out_ref):
    h = jnp.maximum(lax.dot_general(enc_ref[...], w1_ref[...],
                                    (((0,), (0,)), ((), ()))), 0.0)
    h = jnp.maximum(jnp.dot(h, w2_ref[...]), 0.0)
    out_ref[...] = jnp.dot(h, w3_ref[...])


@jax.jit
def _mlp(enc, W1, W2, W3):
    BN = 4096
    return pl.pallas_call(
        _mlp_body,
        grid=(N // BN,),
        in_specs=[
            pl.BlockSpec((2 * LVLS, BN), lambda i: (0, i)),
            pl.BlockSpec((2 * LVLS, 64), lambda i: (0, 0)),
            pl.BlockSpec((64, 64), lambda i: (0, 0)),
            pl.BlockSpec((64, 1), lambda i: (0, 0)),
        ],
        out_specs=pl.BlockSpec((BN, 1), lambda i: (i, 0)),
        out_shape=jax.ShapeDtypeStruct((N, 1), jnp.float32),
    )(enc, W1, W2, W3)


def kernel(xyz, t, tables, W1, W2, W3):
    xyzt_t = jnp.concatenate([xyz, t], axis=1).T
    tab0 = tables[:, :, 0].reshape(LVLS * TSIZE)
    tab1 = tables[:, :, 1].reshape(LVLS * TSIZE)
    enc = _encode(xyzt_t, tab0, tab1)
    return _mlp(enc, W1, W2, W3)


# speedup vs baseline: 111.4433x; 1.1268x over previous
import numpy as np
import jax
import jax.numpy as jnp
from jax import lax
from jax.experimental import pallas as pl
from jax.experimental.pallas import tpu as pltpu
from jax.experimental.pallas import tpu_sc as plsc

LVLS = 16
TSIZE = 2 ** 19
HMASK = TSIZE - 1
P1, P2, P3 = 2654435761, 805459861, 3674653429

N = 524288
NC, NS, LANES = 2, 16, 16
NW = NC * NS
PER_TILE = N // NW
C = 512
NCHUNK = PER_TILE // C
NG = C // LANES
K = 16 * C


def _enc_body(xyzt_hbm, tab0_hbm, tab1_hbm, enc_hbm,
              xbuf, idx0, idx1, w0, w1, g00, g10, g01, g11, acc_buf,
              sem0, sem1):
    wid = lax.axis_index("s") * NC + lax.axis_index("c")
    tile_base = wid * PER_TILE

    def hash_level(res, ltv, idxr, wr):
        def hash_group(g, _):
            gb = g * LANES
            x = xbuf[0, pl.ds(gb, LANES)]
            y = xbuf[1, pl.ds(gb, LANES)]
            z = xbuf[2, pl.ds(gb, LANES)]
            t = xbuf[3, pl.ds(gb, LANES)]
            hs = []
            ws = []
            for coord, prime in ((x, 1), (y, P1), (z, P2), (t, P3)):
                pos = coord * res
                p0 = pos.astype(jnp.int32)
                frac = pos - p0.astype(jnp.float32)
                u0 = p0.astype(jnp.uint32)
                pj = jnp.uint32(prime)
                h0 = u0 * pj
                hs.append((h0, h0 + pj))
                ws.append((1.0 - frac, frac))
            hxy = [[hs[0][a] ^ hs[1][b] for b in (0, 1)] for a in (0, 1)]
            hzt = [[hs[2][a] ^ hs[3][b] for b in (0, 1)] for a in (0, 1)]
            wxy = [[ws[0][a] * ws[1][b] for b in (0, 1)] for a in (0, 1)]
            wzt = [[ws[2][a] * ws[3][b] for b in (0, 1)] for a in (0, 1)]
            for c in range(16):
                bx, by, bz, bt = c & 1, (c >> 1) & 1, (c >> 2) & 1, (c >> 3) & 1
                hh = (hxy[bx][by] ^ hzt[bz][bt]) & jnp.uint32(HMASK)
                idxr[pl.ds(c * C + gb, LANES)] = hh.astype(jnp.int32) + ltv
                wr[pl.ds(c * C + gb, LANES)] = wxy[bx][by] * wzt[bz][bt]
            return 0

        lax.fori_loop(0, NG, hash_group, 0)

    def fire(idxr, g0r, g1r, semr):
        pltpu.async_copy(tab0_hbm.at[idxr], g0r, semr)
        pltpu.async_copy(tab1_hbm.at[idxr], g1r, semr)

    def drain(idxr, g0r, g1r, semr):
        pltpu.make_async_copy(tab0_hbm.at[idxr], g0r, semr).wait()
        pltpu.make_async_copy(tab1_hbm.at[idxr], g1r, semr).wait()

    def acc_level(l, wr, g0r, g1r):
        def acc_group(g, _):
            gb = g * LANES
            acc0 = jnp.zeros((LANES,), jnp.float32)
            acc1 = jnp.zeros((LANES,), jnp.float32)
            for c in range(16):
                wv = wr[pl.ds(c * C + gb, LANES)]
                acc0 = acc0 + g0r[pl.ds(c * C + gb, LANES)] * wv
                acc1 = acc1 + g1r[pl.ds(c * C + gb, LANES)] * wv
            acc_buf[2 * l, pl.ds(gb, LANES)] = acc0
            acc_buf[2 * l + 1, pl.ds(gb, LANES)] = acc1
            return 0

        lax.fori_loop(0, NG, acc_group, 0)

    def trunc(v):
        return v.astype(jnp.int32).astype(jnp.float32)

    def chunk_body(ci, _):
        base = tile_base + ci * C
        pltpu.sync_copy(xyzt_hbm.at[:, pl.ds(base, C)], xbuf)

        rf0 = jnp.full((LANES,), 16.0, jnp.float32)
        ltv0 = jnp.zeros((LANES,), jnp.int32)
        hash_level(rf0, ltv0, idx0, w0)
        fire(idx0, g00, g10, sem0)

        def body(i, carry):
            rf, ltv = carry
            hash_level(trunc(rf), ltv, idx1, w1)
            fire(idx1, g01, g11, sem1)
            drain(idx0, g00, g10, sem0)
            acc_level(2 * i, w0, g00, g10)
            rf2 = rf * 1.5
            ltv2 = ltv + TSIZE
            hash_level(trunc(rf2), ltv2, idx0, w0)
            fire(idx0, g00, g10, sem0)
            drain(idx1, g01, g11, sem1)
            acc_level(2 * i + 1, w1, g01, g11)
            return rf2 * 1.5, ltv2 + TSIZE

        rfF, ltvF = lax.fori_loop(0, (LVLS - 2) // 2, body,
                                  (rf0 * 1.5, ltv0 + TSIZE))
        hash_level(trunc(rfF), ltvF, idx1, w1)
        fire(idx1, g01, g11, sem1)
        drain(idx0, g00, g10, sem0)
        acc_level(LVLS - 2, w0, g00, g10)
        drain(idx1, g01, g11, sem1)
        acc_level(LVLS - 1, w1, g01, g11)
        pltpu.sync_copy(acc_buf, enc_hbm.at[:, pl.ds(base, C)])
        return 0

    lax.fori_loop(0, NCHUNK, chunk_body, 0)


@jax.jit
def _encode(xyzt_t, tab0, tab1):
    mesh = plsc.VectorSubcoreMesh(core_axis_name="c", subcore_axis_name="s")
    f = pl.kernel(
        _enc_body,
        out_type=jax.ShapeDtypeStruct((2 * LVLS, N), jnp.float32),
        mesh=mesh,
        scratch_types=[
            pltpu.VMEM((4, C), jnp.float32),
            pltpu.VMEM((K,), jnp.int32),
            pltpu.VMEM((K,), jnp.int32),
            pltpu.VMEM((K,), jnp.float32),
            pltpu.VMEM((K,), jnp.float32),
            pltpu.VMEM((K,), jnp.float32),
            pltpu.VMEM((K,), jnp.float32),
            pltpu.VMEM((K,), jnp.float32),
            pltpu.VMEM((K,), jnp.float32),
            pltpu.VMEM((2 * LVLS, C), jnp.float32),
            pltpu.SemaphoreType.DMA,
            pltpu.SemaphoreType.DMA,
        ],
    )
    return f(xyzt_t, tab0, tab1)


def _mlp_body(enc_ref, w1_ref, w2_ref, w3_ref, out_ref):
    h = jnp.maximum(lax.dot_general(enc_ref[...], w1_ref[...],
                                    (((0,), (0,)), ((), ()))), 0.0)
    h = jnp.maximum(jnp.dot(h, w2_ref[...]), 0.0)
    out_ref[...] = jnp.dot(h, w3_ref[...])


@jax.jit
def _mlp(enc, W1, W2, W3):
    BN = 4096
    return pl.pallas_call(
        _mlp_body,
        grid=(N // BN,),
        in_specs=[
            pl.BlockSpec((2 * LVLS, BN), lambda i: (0, i)),
            pl.BlockSpec((2 * LVLS, 64), lambda i: (0, 0)),
            pl.BlockSpec((64, 64), lambda i: (0, 0)),
            pl.BlockSpec((64, 1), lambda i: (0, 0)),
        ],
        out_specs=pl.BlockSpec((BN, 1), lambda i: (i, 0)),
        out_shape=jax.ShapeDtypeStruct((N, 1), jnp.float32),
    )(enc, W1, W2, W3)


def kernel(xyz, t, tables, W1, W2, W3):
    xyzt_t = jnp.concatenate([xyz, t], axis=1).T
    tab0 = tables[:, :, 0].reshape(LVLS * TSIZE)
    tab1 = tables[:, :, 1].reshape(LVLS * TSIZE)
    enc = _encode(xyzt_t, tab0, tab1)
    return _mlp(enc, W1, W2, W3)


# speedup vs baseline: 206.9002x; 1.8566x over previous
import numpy as np
import jax
import jax.numpy as jnp
from jax import lax
from jax.experimental import pallas as pl
from jax.experimental.pallas import tpu as pltpu
from jax.experimental.pallas import tpu_sc as plsc

LVLS = 16
TSIZE = 2 ** 19
HMASK = TSIZE - 1
P1, P2, P3 = 2654435761, 805459861, 3674653429

N = 524288
NC, NS, LANES = 2, 16, 16
NW = NC * NS
PER_TILE = N // NW
C = 512
NCHUNK = PER_TILE // C
NG = C // LANES
K = 16 * C


def _enc_body(xyzt_hbm, tab_hbm, enc_hbm,
              xbuf, idx0, idx1, w0, w1, g0, g1, acc_buf, sem0, sem1):
    wid = lax.axis_index("s") * NC + lax.axis_index("c")
    tile_base = wid * PER_TILE
    umask = jnp.uint32(HMASK)

    def hash_level(res, ltv, idxr, wr):
        def hash_group(g, _):
            gb = g * LANES
            x = xbuf[0, pl.ds(gb, LANES)]
            y = xbuf[1, pl.ds(gb, LANES)]
            z = xbuf[2, pl.ds(gb, LANES)]
            t = xbuf[3, pl.ds(gb, LANES)]
            hs = []
            ws = []
            for coord, prime in ((x, 1), (y, P1), (z, P2), (t, P3)):
                pos = coord * res
                p0 = pos.astype(jnp.int32)
                frac = pos - p0.astype(jnp.float32)
                u0 = p0.astype(jnp.uint32)
                pj = jnp.uint32(prime)
                h0 = u0 * pj
                hs.append((h0, h0 + pj))
                ws.append((1.0 - frac, frac))
            hxy = [[(hs[0][a] ^ hs[1][b]) & umask for b in (0, 1)]
                   for a in (0, 1)]
            hzt = [[(hs[2][a] ^ hs[3][b]) & umask for b in (0, 1)]
                   for a in (0, 1)]
            wxy = [[ws[0][a] * ws[1][b] for b in (0, 1)] for a in (0, 1)]
            wzt = [[ws[2][a] * ws[3][b] for b in (0, 1)] for a in (0, 1)]
            for c in range(16):
                bx, by, bz, bt = c & 1, (c >> 1) & 1, (c >> 2) & 1, (c >> 3) & 1
                hh = hxy[bx][by] ^ hzt[bz][bt]
                idxr[pl.ds(c * C + gb, LANES)] = hh.astype(jnp.int32) + ltv
                wr[pl.ds(c * C + gb, LANES)] = wxy[bx][by] * wzt[bz][bt]
            return 0

        lax.fori_loop(0, NG, hash_group, 0)

    def fire(idxr, gr, semr):
        pltpu.async_copy(tab_hbm.at[idxr], gr, semr)

    def drain(idxr, gr, semr):
        pltpu.make_async_copy(tab_hbm.at[idxr], gr, semr).wait()

    def acc_level(l, wr, gr):
        def acc_group(g, _):
            gb = g * LANES
            acc0 = jnp.zeros((LANES,), jnp.float32)
            acc1 = jnp.zeros((LANES,), jnp.float32)
            for c in range(16):
                v = gr[pl.ds(c * C + gb, LANES)]
                f0 = lax.bitcast_convert_type(v << 16, jnp.float32)
                f1 = lax.bitcast_convert_type(v & jnp.int32(-65536),
                                              jnp.float32)
                wv = wr[pl.ds(c * C + gb, LANES)]
                acc0 = acc0 + f0 * wv
                acc1 = acc1 + f1 * wv
            acc_buf[2 * l, pl.ds(gb, LANES)] = acc0
            acc_buf[2 * l + 1, pl.ds(gb, LANES)] = acc1
            return 0

        lax.fori_loop(0, NG, acc_group, 0)

    def trunc(v):
        return v.astype(jnp.int32).astype(jnp.float32)

    def chunk_body(ci, _):
        base = tile_base + ci * C
        pltpu.sync_copy(xyzt_hbm.at[:, pl.ds(base, C)], xbuf)

        rf0 = jnp.full((LANES,), 16.0, jnp.float32)
        ltv0 = jnp.zeros((LANES,), jnp.int32)
        hash_level(rf0, ltv0, idx0, w0)
        fire(idx0, g0, sem0)

        def body(i, carry):
            rf, ltv = carry
            hash_level(trunc(rf), ltv, idx1, w1)
            fire(idx1, g1, sem1)
            drain(idx0, g0, sem0)
            acc_level(2 * i, w0, g0)
            rf2 = rf * 1.5
            ltv2 = ltv + TSIZE
            hash_level(trunc(rf2), ltv2, idx0, w0)
            fire(idx0, g0, sem0)
            drain(idx1, g1, sem1)
            acc_level(2 * i + 1, w1, g1)
            return rf2 * 1.5, ltv2 + TSIZE

        rfF, ltvF = lax.fori_loop(0, (LVLS - 2) // 2, body,
                                  (rf0 * 1.5, ltv0 + TSIZE))
        hash_level(trunc(rfF), ltvF, idx1, w1)
        fire(idx1, g1, sem1)
        drain(idx0, g0, sem0)
        acc_level(LVLS - 2, w0, g0)
        drain(idx1, g1, sem1)
        acc_level(LVLS - 1, w1, g1)
        pltpu.sync_copy(acc_buf, enc_hbm.at[:, pl.ds(base, C)])
        return 0

    lax.fori_loop(0, NCHUNK, chunk_body, 0)


@jax.jit
def _encode(xyzt_t, tab_packed):
    mesh = plsc.VectorSubcoreMesh(core_axis_name="c", subcore_axis_name="s")
    f = pl.kernel(
        _enc_body,
        out_type=jax.ShapeDtypeStruct((2 * LVLS, N), jnp.float32),
        mesh=mesh,
        scratch_types=[
            pltpu.VMEM((4, C), jnp.float32),
            pltpu.VMEM((K,), jnp.int32),
            pltpu.VMEM((K,), jnp.int32),
            pltpu.VMEM((K,), jnp.float32),
            pltpu.VMEM((K,), jnp.float32),
            pltpu.VMEM((K,), jnp.int32),
            pltpu.VMEM((K,), jnp.int32),
            pltpu.VMEM((2 * LVLS, C), jnp.float32),
            pltpu.SemaphoreType.DMA,
            pltpu.SemaphoreType.DMA,
        ],
    )
    return f(xyzt_t, tab_packed)


def _mlp_body(enc_ref, w1_ref, w2_ref, w3_ref, out_ref):
    h = jnp.maximum(lax.dot_general(enc_ref[...], w1_ref[...],
                                    (((0,), (0,)), ((), ()))), 0.0)
    h = jnp.maximum(jnp.dot(h, w2_ref[...]), 0.0)
    out_ref[...] = jnp.dot(h, w3_ref[...])


@jax.jit
def _mlp(enc, W1, W2, W3):
    BN = 4096
    return pl.pallas_call(
        _mlp_body,
        grid=(N // BN,),
        in_specs=[
            pl.BlockSpec((2 * LVLS, BN), lambda i: (0, i)),
            pl.BlockSpec((2 * LVLS, 64), lambda i: (0, 0)),
            pl.BlockSpec((64, 64), lambda i: (0, 0)),
            pl.BlockSpec((64, 1), lambda i: (0, 0)),
        ],
        out_specs=pl.BlockSpec((BN, 1), lambda i: (i, 0)),
        out_shape=jax.ShapeDtypeStruct((N, 1), jnp.float32),
    )(enc, W1, W2, W3)


def kernel(xyz, t, tables, W1, W2, W3):
    xyzt_t = jnp.concatenate([xyz, t], axis=1).T
    tab_packed = lax.bitcast_convert_type(
        tables.astype(jnp.bfloat16), jnp.int32).reshape(LVLS * TSIZE)
    enc = _encode(xyzt_t, tab_packed)
    return _mlp(enc, W1, W2, W3)


# speedup vs baseline: 207.5541x; 1.0032x over previous
import numpy as np
import jax
import jax.numpy as jnp
from jax import lax
from jax.experimental import pallas as pl
from jax.experimental.pallas import tpu as pltpu
from jax.experimental.pallas import tpu_sc as plsc

LVLS = 16
TSIZE = 2 ** 19
HMASK = TSIZE - 1
P1, P2, P3 = 2654435761, 805459861, 3674653429

N = 524288
NC, NS, LANES = 2, 16, 16
NW = NC * NS
PER_TILE = N // NW
C = 512
NCHUNK = PER_TILE // C
NG = C // LANES
K = 16 * C


def _enc_body(xyzt_hbm, tab_hbm, enc_hbm,
              xbuf, idx0, idx1, w0, w1, g0, g1, acc_buf, sem0, sem1):
    wid = lax.axis_index("s") * NC + lax.axis_index("c")
    tile_base = wid * PER_TILE
    umask = jnp.uint32(HMASK)

    def hash_level(res, ltv, idxr, wr):
        def hash_group(g, _):
            gb = g * LANES
            x = xbuf[0, pl.ds(gb, LANES)]
            y = xbuf[1, pl.ds(gb, LANES)]
            z = xbuf[2, pl.ds(gb, LANES)]
            t = xbuf[3, pl.ds(gb, LANES)]
            hs = []
            ws = []
            for coord, prime in ((x, 1), (y, P1), (z, P2), (t, P3)):
                pos = coord * res
                p0 = pos.astype(jnp.int32)
                frac = pos - p0.astype(jnp.float32)
                u0 = p0.astype(jnp.uint32)
                pj = jnp.uint32(prime)
                h0 = u0 * pj
                hs.append((h0, h0 + pj))
                ws.append((1.0 - frac, frac))
            lu = lax.bitcast_convert_type(ltv, jnp.uint32)
            hxy = [[(hs[0][a] ^ hs[1][b]) & umask for b in (0, 1)]
                   for a in (0, 1)]
            hzt = [[((hs[2][a] ^ hs[3][b]) & umask) ^ lu for b in (0, 1)]
                   for a in (0, 1)]
            wxy = [[ws[0][a] * ws[1][b] for b in (0, 1)] for a in (0, 1)]
            wzt = [[ws[2][a] * ws[3][b] for b in (0, 1)] for a in (0, 1)]
            for c in range(16):
                bx, by, bz, bt = c & 1, (c >> 1) & 1, (c >> 2) & 1, (c >> 3) & 1
                hh = hxy[bx][by] ^ hzt[bz][bt]
                idxr[pl.ds(c * C + gb, LANES)] = lax.bitcast_convert_type(
                    hh, jnp.int32)
                wr[pl.ds(c * C + gb, LANES)] = wxy[bx][by] * wzt[bz][bt]
            return 0

        lax.fori_loop(0, NG, hash_group, 0, unroll=2)

    def fire(idxr, gr, semr):
        pltpu.async_copy(tab_hbm.at[idxr], gr, semr)

    def drain(idxr, gr, semr):
        pltpu.make_async_copy(tab_hbm.at[idxr], gr, semr).wait()

    def acc_level(l, wr, gr):
        def acc_group(g, _):
            gb = g * LANES
            acc0 = jnp.zeros((LANES,), jnp.float32)
            acc1 = jnp.zeros((LANES,), jnp.float32)
            for c in range(16):
                v = gr[pl.ds(c * C + gb, LANES)]
                f0 = lax.bitcast_convert_type(v << 16, jnp.float32)
                f1 = lax.bitcast_convert_type(v & jnp.int32(-65536),
                                              jnp.float32)
                wv = wr[pl.ds(c * C + gb, LANES)]
                acc0 = acc0 + f0 * wv
                acc1 = acc1 + f1 * wv
            acc_buf[2 * l, pl.ds(gb, LANES)] = acc0
            acc_buf[2 * l + 1, pl.ds(gb, LANES)] = acc1
            return 0

        lax.fori_loop(0, NG, acc_group, 0, unroll=2)

    def trunc(v):
        return v.astype(jnp.int32).astype(jnp.float32)

    def chunk_body(ci, _):
        base = tile_base + ci * C
        pltpu.sync_copy(xyzt_hbm.at[:, pl.ds(base, C)], xbuf)

        rf0 = jnp.full((LANES,), 16.0, jnp.float32)
        ltv0 = jnp.zeros((LANES,), jnp.int32)
        hash_level(rf0, ltv0, idx0, w0)
        fire(idx0, g0, sem0)

        def body(i, carry):
            rf, ltv = carry
            hash_level(trunc(rf), ltv, idx1, w1)
            fire(idx1, g1, sem1)
            drain(idx0, g0, sem0)
            acc_level(2 * i, w0, g0)
            rf2 = rf * 1.5
            ltv2 = ltv + TSIZE
            hash_level(trunc(rf2), ltv2, idx0, w0)
            fire(idx0, g0, sem0)
            drain(idx1, g1, sem1)
            acc_level(2 * i + 1, w1, g1)
            return rf2 * 1.5, ltv2 + TSIZE

        rfF, ltvF = lax.fori_loop(0, (LVLS - 2) // 2, body,
                                  (rf0 * 1.5, ltv0 + TSIZE))
        hash_level(trunc(rfF), ltvF, idx1, w1)
        fire(idx1, g1, sem1)
        drain(idx0, g0, sem0)
        acc_level(LVLS - 2, w0, g0)
        drain(idx1, g1, sem1)
        acc_level(LVLS - 1, w1, g1)
        pltpu.sync_copy(acc_buf, enc_hbm.at[:, pl.ds(base, C)])
        return 0

    lax.fori_loop(0, NCHUNK, chunk_body, 0)


@jax.jit
def _encode(xyzt_t, tab_packed):
    mesh = plsc.VectorSubcoreMesh(core_axis_name="c", subcore_axis_name="s")
    f = pl.kernel(
        _enc_body,
        out_type=jax.ShapeDtypeStruct((2 * LVLS, N), jnp.float32),
        mesh=mesh,
        scratch_types=[
            pltpu.VMEM((4, C), jnp.float32),
            pltpu.VMEM((K,), jnp.int32),
            pltpu.VMEM((K,), jnp.int32),
            pltpu.VMEM((K,), jnp.float32),
            pltpu.VMEM((K,), jnp.float32),
            pltpu.VMEM((K,), jnp.int32),
            pltpu.VMEM((K,), jnp.int32),
            pltpu.VMEM((2 * LVLS, C), jnp.float32),
            pltpu.SemaphoreType.DMA,
            pltpu.SemaphoreType.DMA,
        ],
    )
    return f(xyzt_t, tab_packed)


def _mlp_body(enc_ref, w1_ref, w2_ref, w3_ref, out_ref):
    h = jnp.maximum(lax.dot_general(enc_ref[...], w1_ref[...],
                                    (((0,), (0,)), ((), ()))), 0.0)
    h = jnp.maximum(jnp.dot(h, w2_ref[...]), 0.0)
    out_ref[...] = jnp.dot(h, w3_ref[...])


@jax.jit
def _mlp(enc, W1, W2, W3):
    BN = 4096
    return pl.pallas_call(
        _mlp_body,
        grid=(N // BN,),
        in_specs=[
            pl.BlockSpec((2 * LVLS, BN), lambda i: (0, i)),
            pl.BlockSpec((2 * LVLS, 64), lambda i: (0, 0)),
            pl.BlockSpec((64, 64), lambda i: (0, 0)),
            pl.BlockSpec((64, 1), lambda i: (0, 0)),
        ],
        out_specs=pl.BlockSpec((BN, 1), lambda i: (i, 0)),
        out_shape=jax.ShapeDtypeStruct((N, 1), jnp.float32),
    )(enc, W1, W2, W3)


def kernel(xyz, t, tables, W1, W2, W3):
    xyzt_t = jnp.concatenate([xyz, t], axis=1).T
    tab_packed = lax.bitcast_convert_type(
        tables.astype(jnp.bfloat16), jnp.int32).reshape(LVLS * TSIZE)
    enc = _encode(xyzt_t, tab_packed)
    return _mlp(enc, W1, W2, W3)


# speedup vs baseline: 209.9652x; 1.0116x over previous
import numpy as np
import jax
import jax.numpy as jnp
from jax import lax
from jax.experimental import pallas as pl
from jax.experimental.pallas import tpu as pltpu
from jax.experimental.pallas import tpu_sc as plsc

LVLS = 16
TSIZE = 2 ** 19
HMASK = TSIZE - 1
P1, P2, P3 = 2654435761, 805459861, 3674653429

N = 524288
NSLABS = 4
NSLAB = N // NSLABS
NC, NS, LANES = 2, 16, 16
NW = NC * NS
PER_TILE = NSLAB // NW
C = 512
NCHUNK = PER_TILE // C
NG = C // LANES
K = 16 * C


def _make_enc_body(slab):
    base0 = slab * NSLAB

    def _enc_body(xyzt_hbm, tab_hbm, enc_hbm,
                  xbuf, idx0, idx1, w0, w1, g0, g1, acc_buf, sem0, sem1):
        wid = lax.axis_index("s") * NC + lax.axis_index("c")
        tile_base = wid * PER_TILE
        umask = jnp.uint32(HMASK)

        def hash_level(res, ltv, idxr, wr):
            def hash_group(g, _):
                gb = g * LANES
                x = xbuf[0, pl.ds(gb, LANES)]
                y = xbuf[1, pl.ds(gb, LANES)]
                z = xbuf[2, pl.ds(gb, LANES)]
                t = xbuf[3, pl.ds(gb, LANES)]
                hs = []
                ws = []
                for coord, prime in ((x, 1), (y, P1), (z, P2), (t, P3)):
                    pos = coord * res
                    p0 = pos.astype(jnp.int32)
                    frac = pos - p0.astype(jnp.float32)
                    u0 = p0.astype(jnp.uint32)
                    pj = jnp.uint32(prime)
                    h0 = u0 * pj
                    hs.append((h0, h0 + pj))
                    ws.append((1.0 - frac, frac))
                lu = lax.bitcast_convert_type(ltv, jnp.uint32)
                hxy = [[(hs[0][a] ^ hs[1][b]) & umask for b in (0, 1)]
                       for a in (0, 1)]
                hzt = [[((hs[2][a] ^ hs[3][b]) & umask) ^ lu
                        for b in (0, 1)] for a in (0, 1)]
                wxy = [[ws[0][a] * ws[1][b] for b in (0, 1)] for a in (0, 1)]
                wzt = [[ws[2][a] * ws[3][b] for b in (0, 1)] for a in (0, 1)]
                for c in range(16):
                    bx, by = c & 1, (c >> 1) & 1
                    bz, bt = (c >> 2) & 1, (c >> 3) & 1
                    hh = hxy[bx][by] ^ hzt[bz][bt]
                    idxr[pl.ds(c * C + gb, LANES)] = lax.bitcast_convert_type(
                        hh, jnp.int32)
                    wr[pl.ds(c * C + gb, LANES)] = wxy[bx][by] * wzt[bz][bt]
                return 0

            lax.fori_loop(0, NG, hash_group, 0, unroll=2)

        def fire(idxr, gr, semr):
            pltpu.async_copy(tab_hbm.at[idxr], gr, semr)

        def drain(idxr, gr, semr):
            pltpu.make_async_copy(tab_hbm.at[idxr], gr, semr).wait()

        def acc_level(l, wr, gr):
            def acc_group(g, _):
                gb = g * LANES
                acc0 = jnp.zeros((LANES,), jnp.float32)
                acc1 = jnp.zeros((LANES,), jnp.float32)
                for c in range(16):
                    v = gr[pl.ds(c * C + gb, LANES)]
                    f0 = lax.bitcast_convert_type(v << 16, jnp.float32)
                    f1 = lax.bitcast_convert_type(v & jnp.int32(-65536),
                                                  jnp.float32)
                    wv = wr[pl.ds(c * C + gb, LANES)]
                    acc0 = acc0 + f0 * wv
                    acc1 = acc1 + f1 * wv
                acc_buf[2 * l, pl.ds(gb, LANES)] = acc0
                acc_buf[2 * l + 1, pl.ds(gb, LANES)] = acc1
                return 0

            lax.fori_loop(0, NG, acc_group, 0, unroll=2)

        def trunc(v):
            return v.astype(jnp.int32).astype(jnp.float32)

        def chunk_body(ci, _):
            local = tile_base + ci * C
            pltpu.sync_copy(xyzt_hbm.at[:, pl.ds(base0 + local, C)], xbuf)

            rf0 = jnp.full((LANES,), 16.0, jnp.float32)
            ltv0 = jnp.zeros((LANES,), jnp.int32)
            hash_level(rf0, ltv0, idx0, w0)
            fire(idx0, g0, sem0)

            def body(i, carry):
                rf, ltv = carry
                hash_level(trunc(rf), ltv, idx1, w1)
                fire(idx1, g1, sem1)
                drain(idx0, g0, sem0)
                acc_level(2 * i, w0, g0)
                rf2 = rf * 1.5
                ltv2 = ltv + TSIZE
                hash_level(trunc(rf2), ltv2, idx0, w0)
                fire(idx0, g0, sem0)
                drain(idx1, g1, sem1)
                acc_level(2 * i + 1, w1, g1)
                return rf2 * 1.5, ltv2 + TSIZE

            rfF, ltvF = lax.fori_loop(0, (LVLS - 2) // 2, body,
                                      (rf0 * 1.5, ltv0 + TSIZE))
            hash_level(trunc(rfF), ltvF, idx1, w1)
            fire(idx1, g1, sem1)
            drain(idx0, g0, sem0)
            acc_level(LVLS - 2, w0, g0)
            drain(idx1, g1, sem1)
            acc_level(LVLS - 1, w1, g1)
            pltpu.sync_copy(acc_buf, enc_hbm.at[:, pl.ds(local, C)])
            return 0

        lax.fori_loop(0, NCHUNK, chunk_body, 0)

    return _enc_body


def _encode(xyzt_t, tab_packed, slab):
    mesh = plsc.VectorSubcoreMesh(core_axis_name="c", subcore_axis_name="s")
    f = pl.kernel(
        _make_enc_body(slab),
        out_type=jax.ShapeDtypeStruct((2 * LVLS, NSLAB), jnp.float32),
        mesh=mesh,
        scratch_types=[
            pltpu.VMEM((4, C), jnp.float32),
            pltpu.VMEM((K,), jnp.int32),
            pltpu.VMEM((K,), jnp.int32),
            pltpu.VMEM((K,), jnp.float32),
            pltpu.VMEM((K,), jnp.float32),
            pltpu.VMEM((K,), jnp.int32),
            pltpu.VMEM((K,), jnp.int32),
            pltpu.VMEM((2 * LVLS, C), jnp.float32),
            pltpu.SemaphoreType.DMA,
            pltpu.SemaphoreType.DMA,
        ],
    )
    return f(xyzt_t, tab_packed)


def _mlp_body(enc_ref, w1_ref, w2_ref, w3_ref, out_ref):
    h = jnp.maximum(lax.dot_general(enc_ref[...], w1_ref[...],
                                    (((0,), (0,)), ((), ()))), 0.0)
    h = jnp.maximum(jnp.dot(h, w2_ref[...]), 0.0)
    out_ref[...] = jnp.dot(h, w3_ref[...])


def _mlp(enc, W1, W2, W3):
    BN = 4096
    return pl.pallas_call(
        _mlp_body,
        grid=(NSLAB // BN,),
        in_specs=[
            pl.BlockSpec((2 * LVLS, BN), lambda i: (0, i)),
            pl.BlockSpec((2 * LVLS, 64), lambda i: (0, 0)),
            pl.BlockSpec((64, 64), lambda i: (0, 0)),
            pl.BlockSpec((64, 1), lambda i: (0, 0)),
        ],
        out_specs=pl.BlockSpec((BN, 1), lambda i: (i, 0)),
        out_shape=jax.ShapeDtypeStruct((NSLAB, 1), jnp.float32),
    )(enc, W1, W2, W3)


def kernel(xyz, t, tables, W1, W2, W3):
    xyzt_t = jnp.concatenate([xyz, t], axis=1).T
    tab_packed = lax.bitcast_convert_type(
        tables.astype(jnp.bfloat16), jnp.int32).reshape(LVLS * TSIZE)
    outs = []
    for s in range(NSLABS):
        enc = _encode(xyzt_t, tab_packed, s)
        outs.append(_mlp(enc, W1, W2, W3))
    return jnp.concatenate(outs, axis=0)


# speedup vs baseline: 212.7360x; 1.0132x over previous
import numpy as np
import jax
import jax.numpy as jnp
from jax import lax
from jax.experimental import pallas as pl
from jax.experimental.pallas import tpu as pltpu
from jax.experimental.pallas import tpu_sc as plsc

LVLS = 16
TSIZE = 2 ** 19
HMASK = TSIZE - 1
P1, P2, P3 = 2654435761, 805459861, 3674653429

N = 524288
NSLABS = 2
NSLAB = N // NSLABS
NC, NS, LANES = 2, 16, 16
NW = NC * NS
PER_TILE = NSLAB // NW
C = 512
NCHUNK = PER_TILE // C
NG = C // LANES
K = 16 * C


def _make_enc_body(slab):
    base0 = slab * NSLAB

    def _enc_body(xyzt_hbm, tab_hbm, enc_hbm,
                  xbuf, idx0, idx1, w0, w1, g0, g1, acc_buf, sem0, sem1):
        wid = lax.axis_index("s") * NC + lax.axis_index("c")
        tile_base = wid * PER_TILE
        umask = jnp.uint32(HMASK)

        def hash_level(res, ltv, idxr, wr):
            def hash_group(g, _):
                gb = g * LANES
                x = xbuf[0, pl.ds(gb, LANES)]
                y = xbuf[1, pl.ds(gb, LANES)]
                z = xbuf[2, pl.ds(gb, LANES)]
                t = xbuf[3, pl.ds(gb, LANES)]
                hs = []
                ws = []
                for coord, prime in ((x, 1), (y, P1), (z, P2), (t, P3)):
                    pos = coord * res
                    p0 = pos.astype(jnp.int32)
                    frac = pos - p0.astype(jnp.float32)
                    u0 = p0.astype(jnp.uint32)
                    pj = jnp.uint32(prime)
                    h0 = u0 * pj
                    hs.append((h0, h0 + pj))
                    ws.append((1.0 - frac, frac))
                lu = lax.bitcast_convert_type(ltv, jnp.uint32)
                hxy = [[(hs[0][a] ^ hs[1][b]) & umask for b in (0, 1)]
                       for a in (0, 1)]
                hzt = [[((hs[2][a] ^ hs[3][b]) & umask) ^ lu
                        for b in (0, 1)] for a in (0, 1)]
                wxy = [[ws[0][a] * ws[1][b] for b in (0, 1)] for a in (0, 1)]
                wzt = [[ws[2][a] * ws[3][b] for b in (0, 1)] for a in (0, 1)]
                for c in range(16):
                    bx, by = c & 1, (c >> 1) & 1
                    bz, bt = (c >> 2) & 1, (c >> 3) & 1
                    hh = hxy[bx][by] ^ hzt[bz][bt]
                    idxr[pl.ds(c * C + gb, LANES)] = lax.bitcast_convert_type(
                        hh, jnp.int32)
                    wr[pl.ds(c * C + gb, LANES)] = wxy[bx][by] * wzt[bz][bt]
                return 0

            lax.fori_loop(0, NG, hash_group, 0, unroll=2)

        def fire(idxr, gr, semr):
            pltpu.async_copy(tab_hbm.at[idxr], gr, semr)

        def drain(idxr, gr, semr):
            pltpu.make_async_copy(tab_hbm.at[idxr], gr, semr).wait()

        def acc_level(l, wr, gr):
            def acc_group(g, _):
                gb = g * LANES
                acc0 = jnp.zeros((LANES,), jnp.float32)
                acc1 = jnp.zeros((LANES,), jnp.float32)
                for c in range(16):
                    v = gr[pl.ds(c * C + gb, LANES)]
                    f0 = lax.bitcast_convert_type(v << 16, jnp.float32)
                    f1 = lax.bitcast_convert_type(v & jnp.int32(-65536),
                                                  jnp.float32)
                    wv = wr[pl.ds(c * C + gb, LANES)]
                    acc0 = acc0 + f0 * wv
                    acc1 = acc1 + f1 * wv
                acc_buf[2 * l, pl.ds(gb, LANES)] = acc0
                acc_buf[2 * l + 1, pl.ds(gb, LANES)] = acc1
                return 0

            lax.fori_loop(0, NG, acc_group, 0, unroll=2)

        def trunc(v):
            return v.astype(jnp.int32).astype(jnp.float32)

        def chunk_body(ci, _):
            local = tile_base + ci * C
            pltpu.sync_copy(xyzt_hbm.at[:, pl.ds(base0 + local, C)], xbuf)

            rf0 = jnp.full((LANES,), 16.0, jnp.float32)
            ltv0 = jnp.zeros((LANES,), jnp.int32)
            hash_level(rf0, ltv0, idx0, w0)
            fire(idx0, g0, sem0)

            def body(i, carry):
                rf, ltv = carry
                hash_level(trunc(rf), ltv, idx1, w1)
                fire(idx1, g1, sem1)
                drain(idx0, g0, sem0)
                acc_level(2 * i, w0, g0)
                rf2 = rf * 1.5
                ltv2 = ltv + TSIZE
                hash_level(trunc(rf2), ltv2, idx0, w0)
                fire(idx0, g0, sem0)
                drain(idx1, g1, sem1)
                acc_level(2 * i + 1, w1, g1)
                return rf2 * 1.5, ltv2 + TSIZE

            rfF, ltvF = lax.fori_loop(0, (LVLS - 2) // 2, body,
                                      (rf0 * 1.5, ltv0 + TSIZE))
            hash_level(trunc(rfF), ltvF, idx1, w1)
            fire(idx1, g1, sem1)
            drain(idx0, g0, sem0)
            acc_level(LVLS - 2, w0, g0)
            drain(idx1, g1, sem1)
            acc_level(LVLS - 1, w1, g1)
            pltpu.sync_copy(acc_buf, enc_hbm.at[:, pl.ds(local, C)])
            return 0

        lax.fori_loop(0, NCHUNK, chunk_body, 0)

    return _enc_body


def _encode(xyzt_t, tab_packed, slab):
    mesh = plsc.VectorSubcoreMesh(core_axis_name="c", subcore_axis_name="s")
    f = pl.kernel(
        _make_enc_body(slab),
        out_type=jax.ShapeDtypeStruct((2 * LVLS, NSLAB), jnp.float32),
        mesh=mesh,
        scratch_types=[
            pltpu.VMEM((4, C), jnp.float32),
            pltpu.VMEM((K,), jnp.int32),
            pltpu.VMEM((K,), jnp.int32),
            pltpu.VMEM((K,), jnp.float32),
            pltpu.VMEM((K,), jnp.float32),
            pltpu.VMEM((K,), jnp.int32),
            pltpu.VMEM((K,), jnp.int32),
            pltpu.VMEM((2 * LVLS, C), jnp.float32),
            pltpu.SemaphoreType.DMA,
            pltpu.SemaphoreType.DMA,
        ],
    )
    return f(xyzt_t, tab_packed)


def _mlp_body(enc_ref, w1_ref, w2_ref, w3_ref, out_ref):
    h = jnp.maximum(lax.dot_general(enc_ref[...], w1_ref[...],
                                    (((0,), (0,)), ((), ()))), 0.0)
    h = jnp.maximum(jnp.dot(h, w2_ref[...]), 0.0)
    out_ref[...] = jnp.dot(h, w3_ref[...])


def _mlp(enc, W1, W2, W3):
    BN = 4096
    return pl.pallas_call(
        _mlp_body,
        grid=(NSLAB // BN,),
        in_specs=[
            pl.BlockSpec((2 * LVLS, BN), lambda i: (0, i)),
            pl.BlockSpec((2 * LVLS, 64), lambda i: (0, 0)),
            pl.BlockSpec((64, 64), lambda i: (0, 0)),
            pl.BlockSpec((64, 1), lambda i: (0, 0)),
        ],
        out_specs=pl.BlockSpec((BN, 1), lambda i: (i, 0)),
        out_shape=jax.ShapeDtypeStruct((NSLAB, 1), jnp.float32),
    )(enc, W1, W2, W3)


def kernel(xyz, t, tables, W1, W2, W3):
    xyzt_t = jnp.concatenate([xyz, t], axis=1).T
    tab_packed = lax.bitcast_convert_type(
        tables.astype(jnp.bfloat16), jnp.int32).reshape(LVLS * TSIZE)
    outs = []
    for s in range(NSLABS):
        enc = _encode(xyzt_t, tab_packed, s)
        outs.append(_mlp(enc, W1, W2, W3))
    return jnp.concatenate(outs, axis=0)
